# baseline (speedup 1.0000x reference)
import functools

import jax
import jax.numpy as jnp
from jax.experimental import pallas as pl
from jax.experimental.pallas import tpu as pltpu

DIM_STATE_CONT = 23


def _mlp_kernel(xt_ref, w1_ref, b1_ref, wdt_ref, bd_ref, wct_ref, bc_ref,
                disc_ref, mean_ref, std_ref):
    nc = DIM_STATE_CONT
    h = jax.lax.dot_general(
        w1_ref[...], xt_ref[...], (((0,), (0,)), ((), ())),
        preferred_element_type=jnp.float32) + b1_ref[...]
    h = jnp.where(h >= 0, h, 0.01 * h)
    disc_ref[...] = jnp.dot(wdt_ref[...], h,
                            preferred_element_type=jnp.float32) + bd_ref[...]
    cont = jnp.dot(wct_ref[...], h,
                   preferred_element_type=jnp.float32) + bc_ref[...]
    mean_ref[...] = jnp.clip(cont[:nc, :], -1.0, 1.0)
    std_ref[...] = jnp.clip(cont[nc:, :], 0.0, 1.0)


@functools.partial(jax.jit, static_argnames=("block_n",))
def _run(x, W1, b1, W_disc, b_disc, W_cont, b_cont, block_n=4096):
    batch, dim_in = x.shape
    dim_h = W1.shape[1]
    dim_disc = W_disc.shape[1]
    nc = DIM_STATE_CONT

    xt = x.T
    wdt = W_disc.T
    wct = W_cont.T
    b1c = b1.reshape(dim_h, 1)
    bdc = b_disc.reshape(dim_disc, 1)
    bcc = b_cont.reshape(2 * nc, 1)

    grid = (batch // block_n,)
    col_spec = lambda r: pl.BlockSpec((r, block_n), lambda j: (0, j))
    full_spec = lambda r, c: pl.BlockSpec((r, c), lambda j: (0, 0))

    disc_t, mean_t, std_t = pl.pallas_call(
        _mlp_kernel,
        grid=grid,
        in_specs=[
            col_spec(dim_in),
            full_spec(dim_in, dim_h),
            full_spec(dim_h, 1),
            full_spec(dim_disc, dim_h),
            full_spec(dim_disc, 1),
            full_spec(2 * nc, dim_h),
            full_spec(2 * nc, 1),
        ],
        out_specs=[
            col_spec(dim_disc),
            col_spec(nc),
            col_spec(nc),
        ],
        out_shape=[
            jax.ShapeDtypeStruct((dim_disc, batch), jnp.float32),
            jax.ShapeDtypeStruct((nc, batch), jnp.float32),
            jax.ShapeDtypeStruct((nc, batch), jnp.float32),
        ],
        compiler_params=pltpu.CompilerParams(
            dimension_semantics=("arbitrary",),
        ),
    )(xt, W1, b1c, wdt, bdc, wct, bcc)
    return disc_t.T, mean_t.T, std_t.T


def kernel(x, W1, b1, W_disc, b_disc, W_cont, b_cont):
    disc, mean, std = _run(x, W1, b1, W_disc, b_disc, W_cont, b_cont)
    return (disc, mean, std)
